# baseline (speedup 1.0000x reference)
import functools

import jax
import jax.numpy as jnp
from jax import lax
from jax.experimental import pallas as pl
from jax.experimental.pallas import tpu as pltpu
from jax.experimental.pallas import tpu_sc as plsc

VOCAB = 50368
HID = 1024
EPS = 1e-5

NC = 2
NS = 16
NW = NC * NS

N_TOKENS = 4 * 4096
P = 4
STAGE = N_TOKENS // P
TOK_PER_W = STAGE // NW
CHUNK = 32
N_CHUNKS = TOK_PER_W // CHUNK
NBUF = 3

TC_BLK = 512
TC_BLOCKS = STAGE // TC_BLK


def _gather_kernel(ids_hbm, table_hbm, out_hbm,
                   idx_v, rows_v, gs0, gs1, gs2, os0, os1, os2):
    wid = lax.axis_index("s") * NC + lax.axis_index("c")
    base = wid * TOK_PER_W
    gsem = (gs0, gs1, gs2)
    osem = (os0, os1, os2)

    pltpu.sync_copy(ids_hbm.at[pl.ds(base, TOK_PER_W)], idx_v)

    def gather_copy(c, s):
        return pltpu.make_async_copy(
            table_hbm.at[idx_v.at[pl.ds(c * CHUNK, CHUNK)]],
            rows_v.at[s], gsem[s])

    def out_copy(c, s):
        return pltpu.make_async_copy(
            rows_v.at[s], out_hbm.at[pl.ds(base + c * CHUNK, CHUNK)], osem[s])

    for c in range(min(NBUF, N_CHUNKS)):
        gather_copy(c, c).start()

    def round_body(i, _):
        for k in range(NBUF):
            c = NBUF * i + k
            gather_copy(c, k).wait()
            out_copy(c, k).start()

            def _refill(c=c, k=k):
                out_copy(c, k).wait()
                gather_copy(c + NBUF, k).start()

            pl.when(c + NBUF < N_CHUNKS)(_refill)
        return 0

    n_rounds = N_CHUNKS // NBUF
    if n_rounds:
        lax.fori_loop(0, n_rounds, round_body, 0)

    for c in range(NBUF * n_rounds, N_CHUNKS):
        gather_copy(c, c % NBUF).wait()
        out_copy(c, c % NBUF).start()

    for c in range(max(0, N_CHUNKS - NBUF), N_CHUNKS):
        out_copy(c, c % NBUF).wait()


def _sc_gather(ids_stage, table):
    mesh = plsc.VectorSubcoreMesh(core_axis_name="c", subcore_axis_name="s")
    return pl.kernel(
        _gather_kernel,
        out_type=jax.ShapeDtypeStruct((STAGE, HID), jnp.float32),
        mesh=mesh,
        scratch_types=[
            pltpu.VMEM((TOK_PER_W,), jnp.int32),
            pltpu.VMEM((NBUF, CHUNK, HID), jnp.float32),
            pltpu.SemaphoreType.DMA,
            pltpu.SemaphoreType.DMA,
            pltpu.SemaphoreType.DMA,
            pltpu.SemaphoreType.DMA,
            pltpu.SemaphoreType.DMA,
            pltpu.SemaphoreType.DMA,
        ],
    )(ids_stage, table)


def _ln_block(x, g):
    mean = jnp.mean(x, axis=1, keepdims=True)
    xc = x - mean
    var = jnp.mean(xc * xc, axis=1, keepdims=True)
    return xc * lax.rsqrt(var + EPS) * g


def _tc_ln_first(x_ref, g_ref, o_ref):
    o_ref[...] = _ln_block(x_ref[...], g_ref[...])


def _tc_ln_chained(x_ref, g_ref, buf_ref, o_ref):
    del buf_ref
    o_ref[...] = _ln_block(x_ref[...], g_ref[...])


def _tc_layer_norm_stage(emb, gamma2d, buf, k):
    out_spec = pl.BlockSpec(
        (TC_BLK, HID), lambda i, k=k: (k * TC_BLOCKS + i, 0))
    in_specs = [
        pl.BlockSpec((TC_BLK, HID), lambda i: (i, 0)),
        pl.BlockSpec((1, HID), lambda i: (0, 0)),
    ]
    args = [emb, gamma2d]
    if buf is None:
        body = _tc_ln_first
        aliases = {}
    else:
        body = _tc_ln_chained
        in_specs.append(pl.BlockSpec(memory_space=pl.ANY))
        args.append(buf)
        aliases = {2: 0}
    return pl.pallas_call(
        body,
        grid=(TC_BLOCKS,),
        in_specs=in_specs,
        out_specs=out_spec,
        out_shape=jax.ShapeDtypeStruct((N_TOKENS, HID), jnp.float32),
        input_output_aliases=aliases,
    )(*args)


@jax.jit
def kernel(input_ids, table, gamma):
    ids_flat = input_ids.reshape(-1).astype(jnp.int32)
    gamma2d = gamma.reshape(1, HID)
    embs = [_sc_gather(ids_flat[k * STAGE:(k + 1) * STAGE], table)
            for k in range(P)]
    buf = None
    for k in range(P):
        buf = _tc_layer_norm_stage(embs[k], gamma2d, buf, k)
    return buf.reshape(input_ids.shape + (HID,))


# speedup vs baseline: 1.0237x; 1.0237x over previous
import functools

import jax
import jax.numpy as jnp
from jax import lax
from jax.experimental import pallas as pl
from jax.experimental.pallas import tpu as pltpu
from jax.experimental.pallas import tpu_sc as plsc

VOCAB = 50368
HID = 1024
EPS = 1e-5

NC = 2
NS = 16
NW = NC * NS

N_TOKENS = 4 * 4096
P = 4
STAGE = N_TOKENS // P
TOK_PER_W = STAGE // NW
CHUNK = 32
N_CHUNKS = TOK_PER_W // CHUNK
NBUF = 3

TC_BLK = 1024
TC_BLOCKS = STAGE // TC_BLK


def _gather_kernel(ids_hbm, table_hbm, out_hbm,
                   idx_v, rows_v, gs0, gs1, gs2, os0, os1, os2):
    wid = lax.axis_index("s") * NC + lax.axis_index("c")
    base = wid * TOK_PER_W
    gsem = (gs0, gs1, gs2)
    osem = (os0, os1, os2)

    pltpu.sync_copy(ids_hbm.at[pl.ds(base, TOK_PER_W)], idx_v)

    def gather_copy(c, s):
        return pltpu.make_async_copy(
            table_hbm.at[idx_v.at[pl.ds(c * CHUNK, CHUNK)]],
            rows_v.at[s], gsem[s])

    def out_copy(c, s):
        return pltpu.make_async_copy(
            rows_v.at[s], out_hbm.at[pl.ds(base + c * CHUNK, CHUNK)], osem[s])

    for c in range(min(NBUF, N_CHUNKS)):
        gather_copy(c, c).start()

    def round_body(i, _):
        for k in range(NBUF):
            c = NBUF * i + k
            gather_copy(c, k).wait()
            out_copy(c, k).start()

            def _refill(c=c, k=k):
                out_copy(c, k).wait()
                gather_copy(c + NBUF, k).start()

            pl.when(c + NBUF < N_CHUNKS)(_refill)
        return 0

    n_rounds = N_CHUNKS // NBUF
    if n_rounds:
        lax.fori_loop(0, n_rounds, round_body, 0)

    for c in range(NBUF * n_rounds, N_CHUNKS):
        gather_copy(c, c % NBUF).wait()
        out_copy(c, c % NBUF).start()

    for c in range(max(0, N_CHUNKS - NBUF), N_CHUNKS):
        out_copy(c, c % NBUF).wait()


def _sc_gather(ids_stage, table):
    mesh = plsc.VectorSubcoreMesh(core_axis_name="c", subcore_axis_name="s")
    return pl.kernel(
        _gather_kernel,
        out_type=jax.ShapeDtypeStruct((STAGE, HID), jnp.float32),
        mesh=mesh,
        scratch_types=[
            pltpu.VMEM((TOK_PER_W,), jnp.int32),
            pltpu.VMEM((NBUF, CHUNK, HID), jnp.float32),
            pltpu.SemaphoreType.DMA,
            pltpu.SemaphoreType.DMA,
            pltpu.SemaphoreType.DMA,
            pltpu.SemaphoreType.DMA,
            pltpu.SemaphoreType.DMA,
            pltpu.SemaphoreType.DMA,
        ],
    )(ids_stage, table)


def _ln_block(x, g):
    mean = jnp.mean(x, axis=1, keepdims=True)
    xc = x - mean
    var = jnp.mean(xc * xc, axis=1, keepdims=True)
    return xc * lax.rsqrt(var + EPS) * g


def _tc_ln_first(x_ref, g_ref, o_ref):
    o_ref[...] = _ln_block(x_ref[...], g_ref[...])


def _tc_ln_chained(x_ref, g_ref, buf_ref, o_ref):
    del buf_ref
    o_ref[...] = _ln_block(x_ref[...], g_ref[...])


def _tc_layer_norm_stage(emb, gamma2d, buf, k):
    out_spec = pl.BlockSpec(
        (TC_BLK, HID), lambda i, k=k: (k * TC_BLOCKS + i, 0))
    in_specs = [
        pl.BlockSpec((TC_BLK, HID), lambda i: (i, 0)),
        pl.BlockSpec((1, HID), lambda i: (0, 0)),
    ]
    args = [emb, gamma2d]
    if buf is None:
        body = _tc_ln_first
        aliases = {}
    else:
        body = _tc_ln_chained
        in_specs.append(pl.BlockSpec(memory_space=pl.ANY))
        args.append(buf)
        aliases = {2: 0}
    return pl.pallas_call(
        body,
        grid=(TC_BLOCKS,),
        in_specs=in_specs,
        out_specs=out_spec,
        out_shape=jax.ShapeDtypeStruct((N_TOKENS, HID), jnp.float32),
        input_output_aliases=aliases,
    )(*args)


@jax.jit
def kernel(input_ids, table, gamma):
    ids_flat = input_ids.reshape(-1).astype(jnp.int32)
    gamma2d = gamma.reshape(1, HID)
    embs = [_sc_gather(ids_flat[k * STAGE:(k + 1) * STAGE], table)
            for k in range(P)]
    buf = None
    for k in range(P):
        buf = _tc_layer_norm_stage(embs[k], gamma2d, buf, k)
    return buf.reshape(input_ids.shape + (HID,))


# speedup vs baseline: 1.0239x; 1.0003x over previous
import functools

import jax
import jax.numpy as jnp
from jax import lax
from jax.experimental import pallas as pl
from jax.experimental.pallas import tpu as pltpu
from jax.experimental.pallas import tpu_sc as plsc

VOCAB = 50368
HID = 1024
EPS = 1e-5

NC = 2
NS = 16
NW = NC * NS

N_TOKENS = 4 * 4096
P = 4
STAGE = N_TOKENS // P
TOK_PER_W = STAGE // NW
CHUNK = 16
N_CHUNKS = TOK_PER_W // CHUNK
NBUF = 6
PREF = 4
LAG = 2

TC_BLK = 1024
TC_BLOCKS = STAGE // TC_BLK


def _gather_kernel(ids_hbm, table_hbm, out_hbm, idx_v, rows_v,
                   gs0, gs1, gs2, gs3, gs4, gs5,
                   os0, os1, os2, os3, os4, os5):
    wid = lax.axis_index("s") * NC + lax.axis_index("c")
    base = wid * TOK_PER_W
    gsem = (gs0, gs1, gs2, gs3, gs4, gs5)
    osem = (os0, os1, os2, os3, os4, os5)

    pltpu.sync_copy(ids_hbm.at[pl.ds(base, TOK_PER_W)], idx_v)

    def gather_copy(c, s):
        return pltpu.make_async_copy(
            table_hbm.at[idx_v.at[pl.ds(c * CHUNK, CHUNK)]],
            rows_v.at[s], gsem[s])

    def out_copy(c, s):
        return pltpu.make_async_copy(
            rows_v.at[s], out_hbm.at[pl.ds(base + c * CHUNK, CHUNK)], osem[s])

    for c in range(min(PREF, N_CHUNKS)):
        gather_copy(c, c % NBUF).start()

    def step(c, s, guard_refill):
        gather_copy(c, s).wait()
        out_copy(c, s).start()

        def _refill(c=c):
            if c - LAG >= 0:
                out_copy(c - LAG, (c - LAG) % NBUF).wait()
            gather_copy(c + PREF, (c + PREF) % NBUF).start()

        if guard_refill:
            pl.when(c + PREF < N_CHUNKS)(_refill)
        else:
            _refill()

    for c in range(min(LAG, N_CHUNKS)):
        step(c, c % NBUF, c + PREF >= N_CHUNKS)

    n_mid = N_CHUNKS - LAG
    assert n_mid % NBUF == 0, (N_CHUNKS, NBUF, LAG)

    def round_body(i, _):
        for k in range(NBUF):
            c = NBUF * i + k + LAG
            def _step(i=i, k=k):
                cc = NBUF * i + k + LAG
                s = (k + LAG) % NBUF
                gather_copy(cc, s).wait()
                out_copy(cc, s).start()

                def _refill():
                    out_copy(cc - LAG, (s - LAG) % NBUF).wait()
                    gather_copy(cc + PREF, (s + PREF) % NBUF).start()

                pl.when(cc + PREF < N_CHUNKS)(_refill)

            _step()
        return 0

    lax.fori_loop(0, n_mid // NBUF, round_body, 0)

    for c in range(max(0, N_CHUNKS - NBUF), N_CHUNKS):
        out_copy(c, c % NBUF).wait()


def _sc_gather(ids_stage, table):
    mesh = plsc.VectorSubcoreMesh(core_axis_name="c", subcore_axis_name="s")
    return pl.kernel(
        _gather_kernel,
        out_type=jax.ShapeDtypeStruct((STAGE, HID), jnp.float32),
        mesh=mesh,
        scratch_types=[
            pltpu.VMEM((TOK_PER_W,), jnp.int32),
            pltpu.VMEM((NBUF, CHUNK, HID), jnp.float32),
        ] + [pltpu.SemaphoreType.DMA] * (2 * NBUF),
    )(ids_stage, table)


def _ln_block(x, g):
    mean = jnp.mean(x, axis=1, keepdims=True)
    xc = x - mean
    var = jnp.mean(xc * xc, axis=1, keepdims=True)
    return xc * lax.rsqrt(var + EPS) * g


def _tc_ln_first(x_ref, g_ref, o_ref):
    o_ref[...] = _ln_block(x_ref[...], g_ref[...])


def _tc_ln_chained(x_ref, g_ref, buf_ref, o_ref):
    del buf_ref
    o_ref[...] = _ln_block(x_ref[...], g_ref[...])


def _tc_layer_norm_stage(emb, gamma2d, buf, k):
    out_spec = pl.BlockSpec(
        (TC_BLK, HID), lambda i, k=k: (k * TC_BLOCKS + i, 0))
    in_specs = [
        pl.BlockSpec((TC_BLK, HID), lambda i: (i, 0)),
        pl.BlockSpec((1, HID), lambda i: (0, 0)),
    ]
    args = [emb, gamma2d]
    if buf is None:
        body = _tc_ln_first
        aliases = {}
    else:
        body = _tc_ln_chained
        in_specs.append(pl.BlockSpec(memory_space=pl.ANY))
        args.append(buf)
        aliases = {2: 0}
    return pl.pallas_call(
        body,
        grid=(TC_BLOCKS,),
        in_specs=in_specs,
        out_specs=out_spec,
        out_shape=jax.ShapeDtypeStruct((N_TOKENS, HID), jnp.float32),
        input_output_aliases=aliases,
    )(*args)


@jax.jit
def kernel(input_ids, table, gamma):
    ids_flat = input_ids.reshape(-1).astype(jnp.int32)
    gamma2d = gamma.reshape(1, HID)
    embs = [_sc_gather(ids_flat[k * STAGE:(k + 1) * STAGE], table)
            for k in range(P)]
    buf = None
    for k in range(P):
        buf = _tc_layer_norm_stage(embs[k], gamma2d, buf, k)
    return buf.reshape(input_ids.shape + (HID,))
